# baseline (speedup 1.0000x reference)
import functools

import jax
import jax.numpy as jnp
from jax import lax
from jax.experimental import pallas as pl
from jax.experimental.pallas import tpu as pltpu
from jax.experimental.pallas import tpu_sc as plsc

_NUM_CORES = 2
_NUM_SUBCORES = 16
_NUM_WORKERS = _NUM_CORES * _NUM_SUBCORES


def kernel(xx, theta):
    n = xx.shape[-1]
    d = theta.shape[1]
    rows_per_w = n // _NUM_WORKERS

    mesh = plsc.VectorSubcoreMesh(core_axis_name="c", subcore_axis_name="s")

    @functools.partial(
        pl.kernel,
        mesh=mesh,
        out_type=jax.ShapeDtypeStruct((n, d), theta.dtype),
    )
    def gather_rows(theta_hbm, out_hbm):
        wid = lax.axis_index("s") * _NUM_CORES + lax.axis_index("c")
        base = wid * rows_per_w
        pltpu.sync_copy(
            theta_hbm.at[pl.ds(base, rows_per_w)],
            out_hbm.at[pl.ds(base, rows_per_w)],
        )

    return gather_rows(theta)


# speedup vs baseline: 23.3449x; 23.3449x over previous
import functools

import jax
import jax.numpy as jnp
from jax import lax
from jax.experimental import pallas as pl
from jax.experimental.pallas import tpu as pltpu
from jax.experimental.pallas import tpu_sc as plsc

_NUM_CORES = 2
_NUM_SUBCORES = 16
_NUM_WORKERS = _NUM_CORES * _NUM_SUBCORES
_CHUNK_ROWS = 16


def kernel(xx, theta):
    n = xx.shape[-1]
    d = theta.shape[1]
    rows_per_w = n // _NUM_WORKERS
    nchunks = rows_per_w // _CHUNK_ROWS

    mesh = plsc.VectorSubcoreMesh(core_axis_name="c", subcore_axis_name="s")

    @functools.partial(
        pl.kernel,
        mesh=mesh,
        out_type=jax.ShapeDtypeStruct((n, d), theta.dtype),
        scratch_types=[
            pltpu.VMEM((_CHUNK_ROWS, d), theta.dtype),
            pltpu.VMEM((_CHUNK_ROWS, d), theta.dtype),
            pltpu.SemaphoreType.DMA,
            pltpu.SemaphoreType.DMA,
            pltpu.SemaphoreType.DMA,
            pltpu.SemaphoreType.DMA,
        ],
    )
    def gather_rows(theta_hbm, out_hbm, buf0, buf1, si0, si1, so0, so1):
        wid = lax.axis_index("s") * _NUM_CORES + lax.axis_index("c")
        base = wid * rows_per_w
        bufs = (buf0, buf1)
        sin = (si0, si1)
        sout = (so0, so1)
        out_copies = [None] * nchunks
        for g in range(nchunks):
            buf = bufs[g % 2]
            if g >= 2:
                out_copies[g - 2].wait()
            pltpu.async_copy(
                theta_hbm.at[pl.ds(base + g * _CHUNK_ROWS, _CHUNK_ROWS)],
                buf,
                sin[g % 2],
            ).wait()
            out_copies[g] = pltpu.async_copy(
                buf,
                out_hbm.at[pl.ds(base + g * _CHUNK_ROWS, _CHUNK_ROWS)],
                sout[g % 2],
            )
        out_copies[nchunks - 2].wait()
        out_copies[nchunks - 1].wait()

    return gather_rows(theta)
